# baseline (speedup 1.0000x reference)
import functools

import jax
import jax.numpy as jnp
from jax.experimental import pallas as pl
from jax.experimental.pallas import tpu as pltpu

_EPS = 1e-8
_HB = 1024


def _dot_t(a, v):
    return jax.lax.dot_general(
        a, v, dimension_numbers=(((0,), (0,)), ((), ())),
        preferred_element_type=jnp.float32)


def _loss_kernel(xa_ref, xb_ref, ta_ref, tb_ref, wt_ref, b_ref, lo_ref,
                 hi_ref, sl_ref, out_ref, mc_ref, s_ref, acc_ref,
                 *, nb, bseg, r, s_total):
    g = pl.program_id(0)

    @pl.when(g == 0)
    def _init():
        mc_ref[...] = jnp.full((1, r), -1e30, jnp.float32)
        s_ref[...] = jnp.zeros((bseg, r), jnp.float32)
        acc_ref[...] = jnp.zeros((bseg, 4 * r), jnp.float32)

    wt = wt_ref[...]
    bias = b_ref[...]
    pa = jnp.dot(xa_ref[...], wt, preferred_element_type=jnp.float32) + bias
    pb = jnp.dot(xb_ref[...], wt, preferred_element_type=jnp.float32) + bias
    ta = ta_ref[...]
    tb = tb_ref[...]

    iota = jax.lax.broadcasted_iota(jnp.int32, (_HB, 1), 0)
    idx_a = iota + g * (2 * _HB)
    idx_b = idx_a + _HB
    oh_a = ((idx_a >= lo_ref[...]) & (idx_a < hi_ref[...])
            ).astype(jnp.float32)
    oh_b = ((idx_b >= lo_ref[...]) & (idx_b < hi_ref[...])
            ).astype(jnp.float32)

    mc_old = mc_ref[...]
    pmax = jnp.maximum(jnp.max(pa, axis=0, keepdims=True),
                       jnp.max(pb, axis=0, keepdims=True))
    mc = jnp.maximum(mc_old, pmax)
    ea = jnp.exp(pa - mc)
    eb = jnp.exp(pb - mc)
    s_ref[...] = (s_ref[...] * jnp.exp(mc_old - mc)
                  + _dot_t(oh_a, ea) + _dot_t(oh_b, eb))
    mc_ref[...] = mc

    cat_a = jnp.concatenate(
        [ta, ta * pa, pa, jnp.where(ta > 0, ta * jnp.log(ta), 0.0)], axis=1)
    cat_b = jnp.concatenate(
        [tb, tb * pb, pb, jnp.where(tb > 0, tb * jnp.log(tb), 0.0)], axis=1)
    acc_ref[...] += _dot_t(oh_a, cat_a) + _dot_t(oh_b, cat_b)

    @pl.when(g == nb - 1)
    def _finalize():
        mc_f = mc_ref[...]
        s = s_ref[...]
        acc = acc_ref[...]
        T1 = acc[:, :r]
        T2 = acc[:, r:2 * r]
        P1 = acc[:, 2 * r:3 * r]
        L1 = acc[:, 3 * r:]
        slb = jnp.broadcast_to(sl_ref[...], (bseg, r))

        rp = mc_f + jnp.log(s)
        pos = T1 > 0
        safe = jnp.where(pos, T1, 1.0)
        sp_seg = jnp.where(pos, T2 / safe, P1)
        sh1_seg = jnp.where(pos, 1.0, slb)
        shape_dev = jnp.where(pos, 1.0 - (L1 / safe - jnp.log(safe)),
                              slb * (1.0 - jnp.log1p(_EPS)))
        rate_dev = T1 - T1 * jnp.log(T1 + _EPS)
        cells = (s * jnp.exp(mc_f - rp) - sp_seg + rp * sh1_seg
                 + jnp.exp(rp) - T1 * rp - shape_dev - rate_dev)
        out_ref[...] = jnp.sum(cells, axis=(0, 1), keepdims=True) / s_total


def kernel(hidden_states, target, cu_seqlens, W, b):
    s_total, d = hidden_states.shape
    r = W.shape[0]
    bseg = cu_seqlens.shape[0] - 1
    nb = s_total // (2 * _HB)

    wt = W.T
    b2 = b.reshape(1, r)
    cu = cu_seqlens.astype(jnp.int32)
    cu_lo = cu[:bseg].reshape(1, bseg)
    cu_hi = cu[1:].reshape(1, bseg)
    seglens = (cu[1:] - cu[:bseg]).astype(jnp.float32).reshape(bseg, 1)

    out = pl.pallas_call(
        functools.partial(_loss_kernel, nb=nb, bseg=bseg, r=r,
                          s_total=s_total),
        grid=(nb,),
        in_specs=[
            pl.BlockSpec((_HB, d), lambda i: (2 * i, 0)),
            pl.BlockSpec((_HB, d), lambda i: (2 * i + 1, 0)),
            pl.BlockSpec((_HB, r), lambda i: (2 * i, 0)),
            pl.BlockSpec((_HB, r), lambda i: (2 * i + 1, 0)),
            pl.BlockSpec((d, r), lambda i: (0, 0)),
            pl.BlockSpec((1, r), lambda i: (0, 0)),
            pl.BlockSpec((1, bseg), lambda i: (0, 0)),
            pl.BlockSpec((1, bseg), lambda i: (0, 0)),
            pl.BlockSpec((bseg, 1), lambda i: (0, 0)),
        ],
        out_specs=pl.BlockSpec((1, 1), lambda i: (0, 0)),
        out_shape=jax.ShapeDtypeStruct((1, 1), jnp.float32),
        scratch_shapes=[
            pltpu.VMEM((1, r), jnp.float32),
            pltpu.VMEM((bseg, r), jnp.float32),
            pltpu.VMEM((bseg, 4 * r), jnp.float32),
        ],
    )(hidden_states, hidden_states, target, target, wt, b2, cu_lo, cu_hi,
      seglens)
    return out.reshape(())


# speedup vs baseline: 1.0277x; 1.0277x over previous
import functools

import jax
import jax.numpy as jnp
from jax.experimental import pallas as pl
from jax.experimental.pallas import tpu as pltpu

_EPS = 1e-8
_TB = 4096
_HIGH = jax.lax.Precision.DEFAULT


def _dot_t(a, v):
    return jax.lax.dot_general(
        a, v, dimension_numbers=(((0,), (0,)), ((), ())),
        precision=_HIGH, preferred_element_type=jnp.float32)


def _loss_kernel(x_ref, t_ref, wt_ref, b_ref, lo_ref, hi_ref, sl_ref,
                 out_ref, mc_ref, s_ref, acc_ref,
                 *, nb, bseg, r, s_total):
    g = pl.program_id(0)

    @pl.when(g == 0)
    def _init():
        mc_ref[...] = jnp.full((1, r), -1e30, jnp.float32)
        s_ref[...] = jnp.zeros((bseg, r), jnp.float32)
        acc_ref[...] = jnp.zeros((bseg, 4 * r), jnp.float32)

    x = x_ref[...]
    preds = jnp.dot(x, wt_ref[...], precision=_HIGH,
                    preferred_element_type=jnp.float32) + b_ref[...]
    t = t_ref[...]

    idx = jax.lax.broadcasted_iota(jnp.int32, (_TB, 1), 0) + g * _TB
    mask = (idx >= lo_ref[...]) & (idx < hi_ref[...])
    oh = mask.astype(jnp.float32)

    mc_old = mc_ref[...]
    mc = jnp.maximum(mc_old, jnp.max(preds, axis=0, keepdims=True))
    e = jnp.exp(preds - mc)
    s_ref[...] = s_ref[...] * jnp.exp(mc_old - mc) + _dot_t(oh, e)
    mc_ref[...] = mc

    tlogt = jnp.where(t > 0, t * jnp.log(t), 0.0)
    cat = jnp.concatenate([t, t * preds, preds, tlogt], axis=1)
    acc_ref[...] += _dot_t(oh, cat)

    @pl.when(g == nb - 1)
    def _finalize():
        mc_f = mc_ref[...]
        s = s_ref[...]
        acc = acc_ref[...]
        T1 = acc[:, :r]
        T2 = acc[:, r:2 * r]
        P1 = acc[:, 2 * r:3 * r]
        L1 = acc[:, 3 * r:]
        slb = jnp.broadcast_to(sl_ref[...], (bseg, r))

        rp = mc_f + jnp.log(s)
        pos = T1 > 0
        safe = jnp.where(pos, T1, 1.0)
        sp_seg = jnp.where(pos, T2 / safe, P1)
        sh1_seg = jnp.where(pos, 1.0, slb)
        shape_dev = jnp.where(pos, 1.0 - (L1 / safe - jnp.log(safe)),
                              slb * (1.0 - jnp.log1p(_EPS)))
        rate_dev = T1 - T1 * jnp.log(T1 + _EPS)
        cells = (s * jnp.exp(mc_f - rp) - sp_seg + rp * sh1_seg
                 + jnp.exp(rp) - T1 * rp - shape_dev - rate_dev)
        out_ref[...] = jnp.sum(cells, axis=(0, 1), keepdims=True) / s_total


def kernel(hidden_states, target, cu_seqlens, W, b):
    s_total, d = hidden_states.shape
    r = W.shape[0]
    bseg = cu_seqlens.shape[0] - 1
    nb = s_total // _TB

    wt = W.T
    b2 = b.reshape(1, r)
    cu = cu_seqlens.astype(jnp.int32)
    cu_lo = cu[:bseg].reshape(1, bseg)
    cu_hi = cu[1:].reshape(1, bseg)
    seglens = (cu[1:] - cu[:bseg]).astype(jnp.float32).reshape(bseg, 1)

    out = pl.pallas_call(
        functools.partial(_loss_kernel, nb=nb, bseg=bseg, r=r,
                          s_total=s_total),
        grid=(nb,),
        in_specs=[
            pl.BlockSpec((_TB, d), lambda i: (i, 0)),
            pl.BlockSpec((_TB, r), lambda i: (i, 0)),
            pl.BlockSpec((d, r), lambda i: (0, 0)),
            pl.BlockSpec((1, r), lambda i: (0, 0)),
            pl.BlockSpec((1, bseg), lambda i: (0, 0)),
            pl.BlockSpec((1, bseg), lambda i: (0, 0)),
            pl.BlockSpec((bseg, 1), lambda i: (0, 0)),
        ],
        out_specs=pl.BlockSpec((1, 1), lambda i: (0, 0)),
        out_shape=jax.ShapeDtypeStruct((1, 1), jnp.float32),
        scratch_shapes=[
            pltpu.VMEM((1, r), jnp.float32),
            pltpu.VMEM((bseg, r), jnp.float32),
            pltpu.VMEM((bseg, 4 * r), jnp.float32),
        ],
    )(hidden_states, target, wt, b2, cu_lo, cu_hi, seglens)
    return out.reshape(())
